# baseline (speedup 1.0000x reference)
import jax
import jax.numpy as jnp
from jax.experimental import pallas as pl

_D_IN = 1024
_D_H = 512


def _fused_fwd(h_ref, wfc_ref, bfc_ref, wcls_ref, bcls_ref, out_ref):
    x = jnp.dot(h_ref[...], wfc_ref[...], preferred_element_type=jnp.float32)
    x = jnp.maximum(x + bfc_ref[...], 0.0)
    out_ref[...] = (
        jnp.dot(x, wcls_ref[...], preferred_element_type=jnp.float32)
        + bcls_ref[...]
    )


def kernel(h, W_fc, b_fc, W_a, b_a, W_b, b_b, W_c, b_c, W_cls, b_cls):
    h = jnp.squeeze(h)
    n, d_in = h.shape
    n_cls = W_cls.shape[1]
    block_m = 2000
    assert n % block_m == 0
    grid = (n // block_m,)
    return pl.pallas_call(
        _fused_fwd,
        grid=grid,
        in_specs=[
            pl.BlockSpec((block_m, d_in), lambda i: (i, 0)),
            pl.BlockSpec((d_in, _D_H), lambda i: (0, 0)),
            pl.BlockSpec((1, _D_H), lambda i: (0, 0)),
            pl.BlockSpec((_D_H, n_cls), lambda i: (0, 0)),
            pl.BlockSpec((1, n_cls), lambda i: (0, 0)),
        ],
        out_specs=pl.BlockSpec((block_m, n_cls), lambda i: (i, 0)),
        out_shape=jax.ShapeDtypeStruct((n, n_cls), jnp.float32),
    )(h, W_fc, b_fc.reshape(1, _D_H), W_cls, b_cls.reshape(1, n_cls))
